# baseline (speedup 1.0000x reference)
import functools

import jax
import jax.numpy as jnp
from jax import lax
from jax.experimental import pallas as pl
from jax.experimental.pallas import tpu as pltpu
from jax.experimental.pallas import tpu_sc as plsc

B = 16384
NC, NS, L = 2, 16, 16
NW = NC * NS
BPW = B // NW
GROUPS = BPW // L
CHUNK = 8

_mesh = plsc.VectorSubcoreMesh(core_axis_name="c", subcore_axis_name="s")


@functools.partial(
    pl.kernel,
    out_type=(
        jax.ShapeDtypeStruct((B * 5,), jnp.float32),
        jax.ShapeDtypeStruct((B * 3,), jnp.float32),
        jax.ShapeDtypeStruct((B * 2,), jnp.float32),
    ),
    mesh=_mesh,
    scratch_types=(
        pltpu.VMEM((BPW,), jnp.int32),
        pltpu.VMEM((BPW,), jnp.int32),
        pltpu.VMEM((BPW,), jnp.float32),
        pltpu.VMEM((BPW,), jnp.float32),
        pltpu.VMEM((400,), jnp.float32),
        pltpu.VMEM((64,), jnp.float32),
        pltpu.VMEM((96,), jnp.float32),
        pltpu.VMEM((BPW * 5,), jnp.float32),
        pltpu.VMEM((BPW * 3,), jnp.float32),
        pltpu.VMEM((BPW * 2,), jnp.float32),
        pltpu.SemaphoreType.DMA,
        pltpu.SemaphoreType.DMA,
    ),
    compiler_params=pltpu.CompilerParams(needs_layout_passes=False),
)
def _sc_embed(aid_h, iid_h, hp_h, atk_h, taba_h, tabi_h, wb_h,
              outa_h, outi_h, outs_h,
              aid_v, iid_v, hp_v, atk_v, taba_v, tabi_v, wb_v,
              outa_v, outi_v, outs_v, sem, sem2):
    wid = lax.axis_index("s") * NC + lax.axis_index("c")
    base = wid * BPW

    early = [
        pltpu.async_copy(aid_h.at[pl.ds(base, BPW)], aid_v, sem),
        pltpu.async_copy(iid_h.at[pl.ds(base, BPW)], iid_v, sem),
        pltpu.async_copy(taba_h, taba_v, sem),
        pltpu.async_copy(tabi_h, tabi_v, sem),
    ]
    late = [
        pltpu.async_copy(hp_h.at[pl.ds(base, BPW)], hp_v, sem2),
        pltpu.async_copy(atk_h.at[pl.ds(base, BPW)], atk_v, sem2),
        pltpu.async_copy(wb_h, wb_v, sem2),
    ]
    for c in early:
        c.wait()

    iota = lax.iota(jnp.int32, L)
    pa0 = iota * 5
    pi0 = iota * 3
    out_copies = []
    for g in range(GROUPS):
        off = g * L
        aidx = aid_v[pl.ds(off, L)] * 5
        iidx = iid_v[pl.ds(off, L)] * 3
        ga = [plsc.load_gather(taba_v, [aidx + j]) for j in range(5)]
        gi = [plsc.load_gather(tabi_v, [iidx + j]) for j in range(3)]
        pa = pa0 + off * 5
        pi = pi0 + off * 3
        for j in range(5):
            plsc.store_scatter(outa_v, [pa + j], ga[j])
        for j in range(3):
            plsc.store_scatter(outi_v, [pi + j], gi[j])
        if g % CHUNK == CHUNK - 1:
            lo = (g + 1 - CHUNK) * L
            n = CHUNK * L
            out_copies += [
                pltpu.async_copy(outa_v.at[pl.ds(lo * 5, n * 5)],
                                 outa_h.at[pl.ds(base * 5 + lo * 5, n * 5)],
                                 sem),
                pltpu.async_copy(outi_v.at[pl.ds(lo * 3, n * 3)],
                                 outi_h.at[pl.ds(base * 3 + lo * 3, n * 3)],
                                 sem),
            ]

    for c in late:
        c.wait()
    w00 = wb_v[pl.ds(0, L)]
    w01 = wb_v[pl.ds(L, L)]
    w10 = wb_v[pl.ds(2 * L, L)]
    w11 = wb_v[pl.ds(3 * L, L)]
    b0 = wb_v[pl.ds(4 * L, L)]
    b1 = wb_v[pl.ds(5 * L, L)]
    ps0 = iota * 2
    for g in range(GROUPS):
        off = g * L
        h = hp_v[pl.ds(off, L)]
        a = atk_v[pl.ds(off, L)]
        ps = ps0 + off * 2
        plsc.store_scatter(outs_v, [ps], h * w00 + a * w01 + b0)
        plsc.store_scatter(outs_v, [ps + 1], h * w10 + a * w11 + b1)
        if g % CHUNK == CHUNK - 1:
            lo = (g + 1 - CHUNK) * L
            n = CHUNK * L
            out_copies.append(
                pltpu.async_copy(outs_v.at[pl.ds(lo * 2, n * 2)],
                                 outs_h.at[pl.ds(base * 2 + lo * 2, n * 2)],
                                 sem))

    for c in out_copies:
        c.wait()


def kernel(animal_id, item_id, hp, atk, emb_animal, emb_item, W_lin, b_lin):
    taba = emb_animal.reshape(-1)
    tabi = jnp.pad(emb_item.reshape(-1), (0, 4))
    wb = jnp.broadcast_to(
        jnp.concatenate([W_lin.reshape(-1), b_lin])[:, None], (6, L)
    ).reshape(-1)
    outa, outi, outs = _sc_embed(animal_id, item_id, hp, atk, taba, tabi, wb)
    return (outa.reshape(B, 5), outi.reshape(B, 3), outs.reshape(B, 2))


# speedup vs baseline: 1.0154x; 1.0154x over previous
import functools

import jax
import jax.numpy as jnp
from jax import lax
from jax.experimental import pallas as pl
from jax.experimental.pallas import tpu as pltpu
from jax.experimental.pallas import tpu_sc as plsc

B = 16384
NC, NS, L = 2, 16, 16
NW = NC * NS
BPW = B // NW
GROUPS = BPW // L
CHUNK = 8

_mesh = plsc.VectorSubcoreMesh(core_axis_name="c", subcore_axis_name="s")


@functools.partial(
    pl.kernel,
    out_type=(
        jax.ShapeDtypeStruct((B * 5,), jnp.float32),
        jax.ShapeDtypeStruct((B * 3,), jnp.float32),
        jax.ShapeDtypeStruct((B * 2,), jnp.float32),
    ),
    mesh=_mesh,
    scratch_types=(
        pltpu.VMEM((BPW,), jnp.int32),
        pltpu.VMEM((BPW,), jnp.int32),
        pltpu.VMEM((BPW,), jnp.float32),
        pltpu.VMEM((BPW,), jnp.float32),
        pltpu.VMEM((400,), jnp.float32),
        pltpu.VMEM((64,), jnp.float32),
        pltpu.VMEM((96,), jnp.float32),
        pltpu.VMEM((BPW * 5,), jnp.float32),
        pltpu.VMEM((BPW * 3,), jnp.float32),
        pltpu.VMEM((BPW * 2,), jnp.float32),
        pltpu.SemaphoreType.DMA,
    ),
    compiler_params=pltpu.CompilerParams(needs_layout_passes=False),
)
def _sc_embed(aid_h, iid_h, hp_h, atk_h, taba_h, tabi_h, wb_h,
              outa_h, outi_h, outs_h,
              aid_v, iid_v, hp_v, atk_v, taba_v, tabi_v, wb_v,
              outa_v, outi_v, outs_v, sem):
    wid = lax.axis_index("s") * NC + lax.axis_index("c")
    base = wid * BPW

    copies = [
        pltpu.async_copy(aid_h.at[pl.ds(base, BPW)], aid_v, sem),
        pltpu.async_copy(iid_h.at[pl.ds(base, BPW)], iid_v, sem),
        pltpu.async_copy(hp_h.at[pl.ds(base, BPW)], hp_v, sem),
        pltpu.async_copy(atk_h.at[pl.ds(base, BPW)], atk_v, sem),
        pltpu.async_copy(taba_h, taba_v, sem),
        pltpu.async_copy(tabi_h, tabi_v, sem),
        pltpu.async_copy(wb_h, wb_v, sem),
    ]
    for c in copies:
        c.wait()

    w00 = wb_v[pl.ds(0, L)]
    w01 = wb_v[pl.ds(L, L)]
    w10 = wb_v[pl.ds(2 * L, L)]
    w11 = wb_v[pl.ds(3 * L, L)]
    b0 = wb_v[pl.ds(4 * L, L)]
    b1 = wb_v[pl.ds(5 * L, L)]
    iota = lax.iota(jnp.int32, L)

    def group_body(g, carry):
        off = g * L
        pos = iota + off
        aidx = aid_v[pl.ds(off, L)] * 5
        iidx = iid_v[pl.ds(off, L)] * 3
        h = hp_v[pl.ds(off, L)]
        a = atk_v[pl.ds(off, L)]
        ga = [plsc.load_gather(taba_v, [aidx + j]) for j in range(5)]
        gi = [plsc.load_gather(tabi_v, [iidx + j]) for j in range(3)]
        s0 = h * w00 + a * w01 + b0
        s1 = h * w10 + a * w11 + b1
        pa = pos * 5
        pi = pos * 3
        ps = pos * 2
        for j in range(5):
            plsc.store_scatter(outa_v, [pa + j], ga[j])
        for j in range(3):
            plsc.store_scatter(outi_v, [pi + j], gi[j])
        plsc.store_scatter(outs_v, [ps], s0)
        plsc.store_scatter(outs_v, [ps + 1], s1)
        return carry

    out_copies = []
    for c in range(GROUPS // CHUNK):
        lax.fori_loop(c * CHUNK, (c + 1) * CHUNK, group_body, 0)
        lo = c * CHUNK * L
        n = CHUNK * L
        out_copies += [
            pltpu.async_copy(outa_v.at[pl.ds(lo * 5, n * 5)],
                             outa_h.at[pl.ds(base * 5 + lo * 5, n * 5)],
                             sem),
            pltpu.async_copy(outi_v.at[pl.ds(lo * 3, n * 3)],
                             outi_h.at[pl.ds(base * 3 + lo * 3, n * 3)],
                             sem),
            pltpu.async_copy(outs_v.at[pl.ds(lo * 2, n * 2)],
                             outs_h.at[pl.ds(base * 2 + lo * 2, n * 2)],
                             sem),
        ]

    for c in out_copies:
        c.wait()


def kernel(animal_id, item_id, hp, atk, emb_animal, emb_item, W_lin, b_lin):
    taba = emb_animal.reshape(-1)
    tabi = jnp.pad(emb_item.reshape(-1), (0, 4))
    wb = jnp.broadcast_to(
        jnp.concatenate([W_lin.reshape(-1), b_lin])[:, None], (6, L)
    ).reshape(-1)
    outa, outi, outs = _sc_embed(animal_id, item_id, hp, atk, taba, tabi, wb)
    return (outa.reshape(B, 5), outi.reshape(B, 3), outs.reshape(B, 2))


# speedup vs baseline: 1.0164x; 1.0010x over previous
import functools

import jax
import jax.numpy as jnp
from jax import lax
from jax.experimental import pallas as pl
from jax.experimental.pallas import tpu as pltpu
from jax.experimental.pallas import tpu_sc as plsc

B = 16384
NC, NS, L = 2, 16, 16
NW = NC * NS
BPW = B // NW
GROUPS = BPW // L
CHUNK = 8

_mesh = plsc.VectorSubcoreMesh(core_axis_name="c", subcore_axis_name="s")


@functools.partial(
    pl.kernel,
    out_type=(
        jax.ShapeDtypeStruct((B * 5,), jnp.float32),
        jax.ShapeDtypeStruct((B * 3,), jnp.float32),
        jax.ShapeDtypeStruct((B * 2,), jnp.float32),
    ),
    mesh=_mesh,
    scratch_types=(
        pltpu.VMEM((BPW,), jnp.int32),
        pltpu.VMEM((BPW,), jnp.int32),
        pltpu.VMEM((BPW,), jnp.float32),
        pltpu.VMEM((BPW,), jnp.float32),
        pltpu.VMEM((400,), jnp.float32),
        pltpu.VMEM((64,), jnp.float32),
        pltpu.VMEM((96,), jnp.float32),
        pltpu.VMEM((BPW * 5,), jnp.float32),
        pltpu.VMEM((BPW * 3,), jnp.float32),
        pltpu.VMEM((BPW * 2,), jnp.float32),
        pltpu.SemaphoreType.DMA,
    ),
    compiler_params=pltpu.CompilerParams(needs_layout_passes=False),
)
def _sc_embed(aid_h, iid_h, hp_h, atk_h, taba_h, tabi_h, wb_h,
              outa_h, outi_h, outs_h,
              aid_v, iid_v, hp_v, atk_v, taba_v, tabi_v, wb_v,
              outa_v, outi_v, outs_v, sem):
    wid = lax.axis_index("s") * NC + lax.axis_index("c")
    base = wid * BPW

    copies = [
        pltpu.async_copy(aid_h.at[pl.ds(base, BPW)], aid_v, sem),
        pltpu.async_copy(iid_h.at[pl.ds(base, BPW)], iid_v, sem),
        pltpu.async_copy(hp_h.at[pl.ds(base, BPW)], hp_v, sem),
        pltpu.async_copy(atk_h.at[pl.ds(base, BPW)], atk_v, sem),
        pltpu.async_copy(taba_h, taba_v, sem),
        pltpu.async_copy(tabi_h, tabi_v, sem),
        pltpu.async_copy(wb_h, wb_v, sem),
    ]
    for c in copies:
        c.wait()

    w00 = wb_v[pl.ds(0, L)]
    w01 = wb_v[pl.ds(L, L)]
    w10 = wb_v[pl.ds(2 * L, L)]
    w11 = wb_v[pl.ds(3 * L, L)]
    b0 = wb_v[pl.ds(4 * L, L)]
    b1 = wb_v[pl.ds(5 * L, L)]
    iota = lax.iota(jnp.int32, L)

    def group_body(g):
        off = g * L
        pos = iota + off
        aidx = aid_v[pl.ds(off, L)] * 5
        iidx = iid_v[pl.ds(off, L)] * 3
        h = hp_v[pl.ds(off, L)]
        a = atk_v[pl.ds(off, L)]
        ga = [plsc.load_gather(taba_v, [aidx + j]) for j in range(5)]
        gi = [plsc.load_gather(tabi_v, [iidx + j]) for j in range(3)]
        s0 = h * w00 + a * w01 + b0
        s1 = h * w10 + a * w11 + b1
        pa = pos * 5
        pi = pos * 3
        ps = pos * 2
        for j in range(5):
            plsc.store_scatter(outa_v, [pa + j], ga[j])
        for j in range(3):
            plsc.store_scatter(outi_v, [pi + j], gi[j])
        plsc.store_scatter(outs_v, [ps], s0)
        plsc.store_scatter(outs_v, [ps + 1], s1)

    out_copies = []
    for c in range(GROUPS // CHUNK):
        plsc.parallel_loop(c * CHUNK, (c + 1) * CHUNK, unroll=2)(group_body)
        lo = c * CHUNK * L
        n = CHUNK * L
        out_copies += [
            pltpu.async_copy(outa_v.at[pl.ds(lo * 5, n * 5)],
                             outa_h.at[pl.ds(base * 5 + lo * 5, n * 5)],
                             sem),
            pltpu.async_copy(outi_v.at[pl.ds(lo * 3, n * 3)],
                             outi_h.at[pl.ds(base * 3 + lo * 3, n * 3)],
                             sem),
            pltpu.async_copy(outs_v.at[pl.ds(lo * 2, n * 2)],
                             outs_h.at[pl.ds(base * 2 + lo * 2, n * 2)],
                             sem),
        ]

    for c in out_copies:
        c.wait()


def kernel(animal_id, item_id, hp, atk, emb_animal, emb_item, W_lin, b_lin):
    taba = emb_animal.reshape(-1)
    tabi = jnp.pad(emb_item.reshape(-1), (0, 4))
    wb = jnp.broadcast_to(
        jnp.concatenate([W_lin.reshape(-1), b_lin])[:, None], (6, L)
    ).reshape(-1)
    outa, outi, outs = _sc_embed(animal_id, item_id, hp, atk, taba, tabi, wb)
    return (outa.reshape(B, 5), outi.reshape(B, 3), outs.reshape(B, 2))


# speedup vs baseline: 1.0348x; 1.0181x over previous
import functools

import jax
import jax.numpy as jnp
from jax import lax
from jax.experimental import pallas as pl
from jax.experimental.pallas import tpu as pltpu
from jax.experimental.pallas import tpu_sc as plsc

B = 16384
NC, NS, L = 2, 16, 16
NW = NC * NS
BPW = B // NW
GROUPS = BPW // L
CHUNK = 8

_mesh = plsc.VectorSubcoreMesh(core_axis_name="c", subcore_axis_name="s")


@functools.partial(
    pl.kernel,
    out_type=(
        jax.ShapeDtypeStruct((B * 5,), jnp.float32),
        jax.ShapeDtypeStruct((B * 3,), jnp.float32),
        jax.ShapeDtypeStruct((B * 2,), jnp.float32),
    ),
    mesh=_mesh,
    scratch_types=(
        pltpu.VMEM((BPW,), jnp.int32),
        pltpu.VMEM((BPW,), jnp.int32),
        pltpu.VMEM((BPW,), jnp.float32),
        pltpu.VMEM((BPW,), jnp.float32),
        pltpu.VMEM((560,), jnp.float32),
        pltpu.VMEM((BPW * 5,), jnp.float32),
        pltpu.VMEM((BPW * 3,), jnp.float32),
        pltpu.VMEM((BPW * 2,), jnp.float32),
        pltpu.SemaphoreType.DMA,
    ),
    compiler_params=pltpu.CompilerParams(needs_layout_passes=False),
)
def _sc_embed(aid_h, iid_h, hp_h, atk_h, tab_h,
              outa_h, outi_h, outs_h,
              aid_v, iid_v, hp_v, atk_v, tab_v,
              outa_v, outi_v, outs_v, sem):
    wid = lax.axis_index("s") * NC + lax.axis_index("c")
    base = wid * BPW

    copies = [
        pltpu.async_copy(aid_h.at[pl.ds(base, BPW)], aid_v, sem),
        pltpu.async_copy(iid_h.at[pl.ds(base, BPW)], iid_v, sem),
        pltpu.async_copy(hp_h.at[pl.ds(base, BPW)], hp_v, sem),
        pltpu.async_copy(atk_h.at[pl.ds(base, BPW)], atk_v, sem),
        pltpu.async_copy(tab_h, tab_v, sem),
    ]
    for c in copies:
        c.wait()

    w00 = tab_v[pl.ds(464, L)]
    w01 = tab_v[pl.ds(464 + L, L)]
    w10 = tab_v[pl.ds(464 + 2 * L, L)]
    w11 = tab_v[pl.ds(464 + 3 * L, L)]
    b0 = tab_v[pl.ds(464 + 4 * L, L)]
    b1 = tab_v[pl.ds(464 + 5 * L, L)]
    iota = lax.iota(jnp.int32, L)

    def group_body(g):
        off = g * L
        pos = iota + off
        aidx = aid_v[pl.ds(off, L)] * 5
        iidx = iid_v[pl.ds(off, L)] * 3
        h = hp_v[pl.ds(off, L)]
        a = atk_v[pl.ds(off, L)]
        ga = [plsc.load_gather(tab_v, [aidx + j]) for j in range(5)]
        gi = [plsc.load_gather(tab_v, [iidx + (400 + j)]) for j in range(3)]
        s0 = h * w00 + a * w01 + b0
        s1 = h * w10 + a * w11 + b1
        pa = pos * 5
        pi = pos * 3
        ps = pos * 2
        for j in range(5):
            plsc.store_scatter(outa_v, [pa + j], ga[j])
        for j in range(3):
            plsc.store_scatter(outi_v, [pi + j], gi[j])
        plsc.store_scatter(outs_v, [ps], s0)
        plsc.store_scatter(outs_v, [ps + 1], s1)

    out_copies = []
    for c in range(GROUPS // CHUNK):
        plsc.parallel_loop(c * CHUNK, (c + 1) * CHUNK, unroll=4)(group_body)
        lo = c * CHUNK * L
        n = CHUNK * L
        out_copies += [
            pltpu.async_copy(outa_v.at[pl.ds(lo * 5, n * 5)],
                             outa_h.at[pl.ds(base * 5 + lo * 5, n * 5)],
                             sem),
            pltpu.async_copy(outi_v.at[pl.ds(lo * 3, n * 3)],
                             outi_h.at[pl.ds(base * 3 + lo * 3, n * 3)],
                             sem),
            pltpu.async_copy(outs_v.at[pl.ds(lo * 2, n * 2)],
                             outs_h.at[pl.ds(base * 2 + lo * 2, n * 2)],
                             sem),
        ]

    for c in out_copies:
        c.wait()


def kernel(animal_id, item_id, hp, atk, emb_animal, emb_item, W_lin, b_lin):
    tab = jnp.concatenate([
        emb_animal.reshape(-1),
        jnp.pad(emb_item.reshape(-1), (0, 4)),
        jnp.broadcast_to(
            jnp.concatenate([W_lin.reshape(-1), b_lin])[:, None], (6, L)
        ).reshape(-1),
    ])
    outa, outi, outs = _sc_embed(animal_id, item_id, hp, atk, tab)
    return (outa.reshape(B, 5), outi.reshape(B, 3), outs.reshape(B, 2))


# speedup vs baseline: 1.0361x; 1.0013x over previous
import functools

import jax
import jax.numpy as jnp
from jax import lax
from jax.experimental import pallas as pl
from jax.experimental.pallas import tpu as pltpu
from jax.experimental.pallas import tpu_sc as plsc

B = 16384
NC, NS, L = 2, 16, 16
NW = NC * NS
BPW = B // NW
GROUPS = BPW // L
CHUNK = 8

_mesh = plsc.VectorSubcoreMesh(core_axis_name="c", subcore_axis_name="s")


@functools.partial(
    pl.kernel,
    out_type=(
        jax.ShapeDtypeStruct((B * 5,), jnp.float32),
        jax.ShapeDtypeStruct((B * 3,), jnp.float32),
        jax.ShapeDtypeStruct((B * 2,), jnp.float32),
    ),
    mesh=_mesh,
    scratch_types=(
        pltpu.VMEM((BPW,), jnp.int32),
        pltpu.VMEM((BPW,), jnp.int32),
        pltpu.VMEM((BPW,), jnp.float32),
        pltpu.VMEM((BPW,), jnp.float32),
        pltpu.VMEM((560,), jnp.float32),
        pltpu.VMEM((BPW * 5,), jnp.float32),
        pltpu.VMEM((BPW * 3,), jnp.float32),
        pltpu.VMEM((BPW * 2,), jnp.float32),
        pltpu.SemaphoreType.DMA,
    ),
    compiler_params=pltpu.CompilerParams(needs_layout_passes=False),
)
def _sc_embed(aid_h, iid_h, hp_h, atk_h, tab_h,
              outa_h, outi_h, outs_h,
              aid_v, iid_v, hp_v, atk_v, tab_v,
              outa_v, outi_v, outs_v, sem):
    wid = lax.axis_index("s") * NC + lax.axis_index("c")
    base = wid * BPW

    copies = [
        pltpu.async_copy(aid_h.at[pl.ds(base, BPW)], aid_v, sem),
        pltpu.async_copy(iid_h.at[pl.ds(base, BPW)], iid_v, sem),
        pltpu.async_copy(hp_h.at[pl.ds(base, BPW)], hp_v, sem),
        pltpu.async_copy(atk_h.at[pl.ds(base, BPW)], atk_v, sem),
        pltpu.async_copy(tab_h, tab_v, sem),
    ]
    for c in copies:
        c.wait()

    w00 = tab_v[pl.ds(464, L)]
    w01 = tab_v[pl.ds(464 + L, L)]
    w10 = tab_v[pl.ds(464 + 2 * L, L)]
    w11 = tab_v[pl.ds(464 + 3 * L, L)]
    b0 = tab_v[pl.ds(464 + 4 * L, L)]
    b1 = tab_v[pl.ds(464 + 5 * L, L)]
    iota = lax.iota(jnp.int32, L)

    def group_body(g):
        off = g * L
        pos = iota + off
        aidx = aid_v[pl.ds(off, L)] * 5
        iidx = iid_v[pl.ds(off, L)] * 3
        h = hp_v[pl.ds(off, L)]
        a = atk_v[pl.ds(off, L)]
        ga = [plsc.load_gather(tab_v, [aidx + j]) for j in range(5)]
        gi = [plsc.load_gather(tab_v, [iidx + (400 + j)]) for j in range(3)]
        s0 = h * w00 + a * w01 + b0
        s1 = h * w10 + a * w11 + b1
        pa = pos * 5
        pi = pos * 3
        ps = pos * 2
        for j in range(5):
            plsc.store_scatter(outa_v, [pa + j], ga[j])
        for j in range(3):
            plsc.store_scatter(outi_v, [pi + j], gi[j])
        plsc.store_scatter(outs_v, [ps], s0)
        plsc.store_scatter(outs_v, [ps + 1], s1)

    out_copies = []
    for c in range(GROUPS // CHUNK):
        plsc.parallel_loop(c * CHUNK, (c + 1) * CHUNK, unroll=8)(group_body)
        lo = c * CHUNK * L
        n = CHUNK * L
        out_copies += [
            pltpu.async_copy(outa_v.at[pl.ds(lo * 5, n * 5)],
                             outa_h.at[pl.ds(base * 5 + lo * 5, n * 5)],
                             sem),
            pltpu.async_copy(outi_v.at[pl.ds(lo * 3, n * 3)],
                             outi_h.at[pl.ds(base * 3 + lo * 3, n * 3)],
                             sem),
            pltpu.async_copy(outs_v.at[pl.ds(lo * 2, n * 2)],
                             outs_h.at[pl.ds(base * 2 + lo * 2, n * 2)],
                             sem),
        ]

    for c in out_copies:
        c.wait()


def kernel(animal_id, item_id, hp, atk, emb_animal, emb_item, W_lin, b_lin):
    tab = jnp.concatenate([
        emb_animal.reshape(-1),
        jnp.pad(emb_item.reshape(-1), (0, 4)),
        jnp.broadcast_to(
            jnp.concatenate([W_lin.reshape(-1), b_lin])[:, None], (6, L)
        ).reshape(-1),
    ])
    outa, outi, outs = _sc_embed(animal_id, item_id, hp, atk, tab)
    return (outa.reshape(B, 5), outi.reshape(B, 3), outs.reshape(B, 2))


# speedup vs baseline: 1.0404x; 1.0041x over previous
import functools

import jax
import jax.numpy as jnp
from jax import lax
from jax.experimental import pallas as pl
from jax.experimental.pallas import tpu as pltpu
from jax.experimental.pallas import tpu_sc as plsc

B = 16384
NC, NS, L = 2, 16, 16
NW = NC * NS
BPW = B // NW
GROUPS = BPW // L
CHUNK = 16

_mesh = plsc.VectorSubcoreMesh(core_axis_name="c", subcore_axis_name="s")


@functools.partial(
    pl.kernel,
    out_type=(
        jax.ShapeDtypeStruct((B * 5,), jnp.float32),
        jax.ShapeDtypeStruct((B * 3,), jnp.float32),
        jax.ShapeDtypeStruct((B * 2,), jnp.float32),
    ),
    mesh=_mesh,
    scratch_types=(
        pltpu.VMEM((BPW,), jnp.int32),
        pltpu.VMEM((BPW,), jnp.int32),
        pltpu.VMEM((BPW,), jnp.float32),
        pltpu.VMEM((BPW,), jnp.float32),
        pltpu.VMEM((560,), jnp.float32),
        pltpu.VMEM((BPW * 5,), jnp.float32),
        pltpu.VMEM((BPW * 3,), jnp.float32),
        pltpu.VMEM((BPW * 2,), jnp.float32),
        pltpu.SemaphoreType.DMA,
    ),
    compiler_params=pltpu.CompilerParams(needs_layout_passes=False),
)
def _sc_embed(aid_h, iid_h, hp_h, atk_h, tab_h,
              outa_h, outi_h, outs_h,
              aid_v, iid_v, hp_v, atk_v, tab_v,
              outa_v, outi_v, outs_v, sem):
    wid = lax.axis_index("s") * NC + lax.axis_index("c")
    base = wid * BPW

    copies = [
        pltpu.async_copy(aid_h.at[pl.ds(base, BPW)], aid_v, sem),
        pltpu.async_copy(iid_h.at[pl.ds(base, BPW)], iid_v, sem),
        pltpu.async_copy(hp_h.at[pl.ds(base, BPW)], hp_v, sem),
        pltpu.async_copy(atk_h.at[pl.ds(base, BPW)], atk_v, sem),
        pltpu.async_copy(tab_h, tab_v, sem),
    ]
    for c in copies:
        c.wait()

    w00 = tab_v[pl.ds(464, L)]
    w01 = tab_v[pl.ds(464 + L, L)]
    w10 = tab_v[pl.ds(464 + 2 * L, L)]
    w11 = tab_v[pl.ds(464 + 3 * L, L)]
    b0 = tab_v[pl.ds(464 + 4 * L, L)]
    b1 = tab_v[pl.ds(464 + 5 * L, L)]
    iota = lax.iota(jnp.int32, L)

    def group_body(g):
        off = g * L
        pos = iota + off
        aidx = aid_v[pl.ds(off, L)] * 5
        iidx = iid_v[pl.ds(off, L)] * 3
        h = hp_v[pl.ds(off, L)]
        a = atk_v[pl.ds(off, L)]
        ga = [plsc.load_gather(tab_v, [aidx + j]) for j in range(5)]
        gi = [plsc.load_gather(tab_v, [iidx + (400 + j)]) for j in range(3)]
        s0 = h * w00 + a * w01 + b0
        s1 = h * w10 + a * w11 + b1
        pa = pos * 5
        pi = pos * 3
        ps = pos * 2
        for j in range(5):
            plsc.store_scatter(outa_v, [pa + j], ga[j])
        for j in range(3):
            plsc.store_scatter(outi_v, [pi + j], gi[j])
        plsc.store_scatter(outs_v, [ps], s0)
        plsc.store_scatter(outs_v, [ps + 1], s1)

    out_copies = []
    for c in range(GROUPS // CHUNK):
        plsc.parallel_loop(c * CHUNK, (c + 1) * CHUNK, unroll=4)(group_body)
        lo = c * CHUNK * L
        n = CHUNK * L
        out_copies += [
            pltpu.async_copy(outa_v.at[pl.ds(lo * 5, n * 5)],
                             outa_h.at[pl.ds(base * 5 + lo * 5, n * 5)],
                             sem),
            pltpu.async_copy(outi_v.at[pl.ds(lo * 3, n * 3)],
                             outi_h.at[pl.ds(base * 3 + lo * 3, n * 3)],
                             sem),
            pltpu.async_copy(outs_v.at[pl.ds(lo * 2, n * 2)],
                             outs_h.at[pl.ds(base * 2 + lo * 2, n * 2)],
                             sem),
        ]

    for c in out_copies:
        c.wait()


def kernel(animal_id, item_id, hp, atk, emb_animal, emb_item, W_lin, b_lin):
    tab = jnp.concatenate([
        emb_animal.reshape(-1),
        jnp.pad(emb_item.reshape(-1), (0, 4)),
        jnp.broadcast_to(
            jnp.concatenate([W_lin.reshape(-1), b_lin])[:, None], (6, L)
        ).reshape(-1),
    ])
    outa, outi, outs = _sc_embed(animal_id, item_id, hp, atk, tab)
    return (outa.reshape(B, 5), outi.reshape(B, 3), outs.reshape(B, 2))


# speedup vs baseline: 1.0422x; 1.0017x over previous
import functools

import jax
import jax.numpy as jnp
from jax import lax
from jax.experimental import pallas as pl
from jax.experimental.pallas import tpu as pltpu
from jax.experimental.pallas import tpu_sc as plsc

B = 16384
NC, NS, L = 2, 16, 16
NW = NC * NS
BPW = B // NW
GROUPS = BPW // L
CHUNK = 16

_mesh = plsc.VectorSubcoreMesh(core_axis_name="c", subcore_axis_name="s")


@functools.partial(
    pl.kernel,
    out_type=(
        jax.ShapeDtypeStruct((B * 5,), jnp.float32),
        jax.ShapeDtypeStruct((B * 3,), jnp.float32),
        jax.ShapeDtypeStruct((B * 2,), jnp.float32),
    ),
    mesh=_mesh,
    scratch_types=(
        pltpu.VMEM((BPW,), jnp.int32),
        pltpu.VMEM((BPW,), jnp.int32),
        pltpu.VMEM((BPW,), jnp.float32),
        pltpu.VMEM((BPW,), jnp.float32),
        pltpu.VMEM((560,), jnp.float32),
        pltpu.VMEM((BPW * 5,), jnp.float32),
        pltpu.VMEM((BPW * 3,), jnp.float32),
        pltpu.VMEM((BPW * 2,), jnp.float32),
        pltpu.SemaphoreType.DMA,
    ),
    compiler_params=pltpu.CompilerParams(needs_layout_passes=False),
)
def _sc_embed(aid_h, iid_h, hp_h, atk_h, tab_h,
              outa_h, outi_h, outs_h,
              aid_v, iid_v, hp_v, atk_v, tab_v,
              outa_v, outi_v, outs_v, sem):
    wid = lax.axis_index("s") * NC + lax.axis_index("c")
    base = wid * BPW

    copies = [
        pltpu.async_copy(aid_h.at[pl.ds(base, BPW)], aid_v, sem),
        pltpu.async_copy(iid_h.at[pl.ds(base, BPW)], iid_v, sem),
        pltpu.async_copy(hp_h.at[pl.ds(base, BPW)], hp_v, sem),
        pltpu.async_copy(atk_h.at[pl.ds(base, BPW)], atk_v, sem),
        pltpu.async_copy(tab_h, tab_v, sem),
    ]
    for c in copies:
        c.wait()

    w00 = tab_v[pl.ds(464, L)]
    w01 = tab_v[pl.ds(464 + L, L)]
    w10 = tab_v[pl.ds(464 + 2 * L, L)]
    w11 = tab_v[pl.ds(464 + 3 * L, L)]
    b0 = tab_v[pl.ds(464 + 4 * L, L)]
    b1 = tab_v[pl.ds(464 + 5 * L, L)]
    iota = lax.iota(jnp.int32, L)

    def group_body(g):
        off = g * L
        pos = iota + off
        aidx = aid_v[pl.ds(off, L)] * 5
        iidx = iid_v[pl.ds(off, L)] * 3
        h = hp_v[pl.ds(off, L)]
        a = atk_v[pl.ds(off, L)]
        ga = [plsc.load_gather(tab_v, [aidx + j]) for j in range(5)]
        gi = [plsc.load_gather(tab_v, [iidx + (400 + j)]) for j in range(3)]
        s0 = h * w00 + a * w01 + b0
        s1 = h * w10 + a * w11 + b1
        pa = pos * 5
        pi = pos * 3
        ps = pos * 2
        for j in range(5):
            plsc.store_scatter(outa_v, [pa + j], ga[j])
        for j in range(3):
            plsc.store_scatter(outi_v, [pi + j], gi[j])
        plsc.store_scatter(outs_v, [ps], s0)
        plsc.store_scatter(outs_v, [ps + 1], s1)

    out_copies = []
    for c in range(GROUPS // CHUNK):
        plsc.parallel_loop(c * CHUNK, (c + 1) * CHUNK, unroll=8)(group_body)
        lo = c * CHUNK * L
        n = CHUNK * L
        out_copies += [
            pltpu.async_copy(outa_v.at[pl.ds(lo * 5, n * 5)],
                             outa_h.at[pl.ds(base * 5 + lo * 5, n * 5)],
                             sem),
            pltpu.async_copy(outi_v.at[pl.ds(lo * 3, n * 3)],
                             outi_h.at[pl.ds(base * 3 + lo * 3, n * 3)],
                             sem),
            pltpu.async_copy(outs_v.at[pl.ds(lo * 2, n * 2)],
                             outs_h.at[pl.ds(base * 2 + lo * 2, n * 2)],
                             sem),
        ]

    for c in out_copies:
        c.wait()


def kernel(animal_id, item_id, hp, atk, emb_animal, emb_item, W_lin, b_lin):
    tab = jnp.concatenate([
        emb_animal.reshape(-1),
        jnp.pad(emb_item.reshape(-1), (0, 4)),
        jnp.broadcast_to(
            jnp.concatenate([W_lin.reshape(-1), b_lin])[:, None], (6, L)
        ).reshape(-1),
    ])
    outa, outi, outs = _sc_embed(animal_id, item_id, hp, atk, tab)
    return (outa.reshape(B, 5), outi.reshape(B, 3), outs.reshape(B, 2))
